# baseline (speedup 1.0000x reference)
import functools

import jax
import jax.numpy as jnp
from jax import lax
from jax.experimental import pallas as pl
from jax.experimental.pallas import tpu as pltpu
from jax.experimental.pallas import tpu_sc as plsc

VOCAB = 64
HID = 8
BATCH = 32
SEQ = 8192
NTOK = BATCH * SEQ
NW = 32
TOK_PER_W = NTOK // NW
CHUNK = 2048
NCHUNK = TOK_PER_W // CHUNK
NGRP = CHUNK // 16


def _sc_body(ids_hbm, w_hbm, hidden_hbm,
             ids0, ids1, hid0, hid1, tmp_v, w_v,
             sem_i0, sem_i1, sem_o0, sem_o1):
    wid = lax.axis_index("s") * 2 + lax.axis_index("c")
    iota = lax.iota(jnp.int32, 16)
    lane_hi = jnp.right_shift(iota, 3)
    kpat = lax.bitwise_and(iota, 7)
    ids_b = (ids0, ids1)
    hid_b = (hid0, hid1)
    sem_i = (sem_i0, sem_i1)
    sem_o = (sem_o0, sem_o1)

    def tok0(c):
        return wid * TOK_PER_W + c * CHUNK

    pltpu.sync_copy(w_hbm, w_v)

    pltpu.async_copy(ids_hbm.at[pl.ds(tok0(0), CHUNK)], ids0, sem_i0)

    def do_chunk(c, b):
        nb = 1 - b
        pltpu.make_async_copy(
            ids_hbm.at[pl.ds(tok0(c), CHUNK)], ids_b[b], sem_i[b]).wait()

        @pl.when(c >= 2)
        def _():
            pltpu.make_async_copy(
                hid_b[b],
                hidden_hbm.at[pl.ds(tok0(c - 2) * HID, CHUNK * HID)],
                sem_o[b]).wait()

        @pl.when(c + 1 < NCHUNK)
        def _():
            pltpu.async_copy(ids_hbm.at[pl.ds(tok0(c + 1), CHUNK)],
                             ids_b[nb], sem_i[nb])

        def grp(g, carry):
            ids16 = ids_b[b][pl.ds(g * 16, 16)]
            tmp_v[...] = ids16 * HID
            for q in range(8):
                idv = plsc.load_gather(tmp_v, [lane_hi + 2 * q])
                hv = plsc.load_gather(w_v, [idv + kpat])
                hid_b[b][pl.ds(g * 128 + q * 16, 16)] = hv
            return carry

        lax.fori_loop(0, NGRP, grp, None)

        pltpu.async_copy(
            hid_b[b], hidden_hbm.at[pl.ds(tok0(c) * HID, CHUNK * HID)],
            sem_o[b])

    def loop_body(jj, carry):
        do_chunk(2 * jj, 0)
        do_chunk(2 * jj + 1, 1)
        return carry

    lax.fori_loop(0, NCHUNK // 2, loop_body, None)

    for c in (NCHUNK - 2, NCHUNK - 1):
        b = c % 2
        pltpu.make_async_copy(
            hid_b[b], hidden_hbm.at[pl.ds(tok0(c) * HID, CHUNK * HID)],
            sem_o[b]).wait()


@functools.partial(
    pl.kernel,
    out_type=[
        jax.ShapeDtypeStruct((NTOK * HID,), jnp.float32),
    ],
    mesh=plsc.VectorSubcoreMesh(core_axis_name="c", subcore_axis_name="s"),
    compiler_params=pltpu.CompilerParams(needs_layout_passes=False),
    scratch_types=[
        pltpu.VMEM((CHUNK,), jnp.int32),
        pltpu.VMEM((CHUNK,), jnp.int32),
        pltpu.VMEM((CHUNK * HID,), jnp.float32),
        pltpu.VMEM((CHUNK * HID,), jnp.float32),
        pltpu.VMEM((16,), jnp.int32),
        pltpu.VMEM((VOCAB * HID,), jnp.float32),
        pltpu.SemaphoreType.DMA,
        pltpu.SemaphoreType.DMA,
        pltpu.SemaphoreType.DMA,
        pltpu.SemaphoreType.DMA,
    ],
)
def _gather_sc(*refs):
    _sc_body(*refs)


TC_BS = 2048


def _tc_body(ids_ref, w0_ref, logits_ref):
    ids = ids_ref[...]
    w0 = w0_ref[...]
    iotav = lax.broadcasted_iota(jnp.int32, (VOCAB, TC_BS), 0)
    onehot = iotav == ids[None, :]
    h0 = jnp.where(onehot, w0[:, None], 0.0).sum(axis=0, keepdims=True)
    t = jnp.maximum(jnp.round(h0 * 10.0), 0.0)
    idx = t.astype(jnp.int32) % VOCAB
    vals = idx.astype(jnp.float32) / 10.0
    logits_ref[...] = jnp.where(iotav == idx, vals, 0.0)


_tc_expand = pl.pallas_call(
    _tc_body,
    grid=(BATCH, SEQ // TC_BS),
    in_specs=[
        pl.BlockSpec((TC_BS,), lambda b, s: (b * (SEQ // TC_BS) + s,)),
        pl.BlockSpec((VOCAB,), lambda b, s: (0,)),
    ],
    out_specs=pl.BlockSpec((None, VOCAB, TC_BS), lambda b, s: (b, 0, s)),
    out_shape=jax.ShapeDtypeStruct((BATCH, VOCAB, SEQ), jnp.float32),
    compiler_params=pltpu.CompilerParams(
        dimension_semantics=("parallel", "parallel")),
)


def kernel(input_ids, embedding_weight):
    ids = input_ids.astype(jnp.int32).reshape(NTOK)
    w = embedding_weight.astype(jnp.float32)
    (hidden_flat,) = _gather_sc(ids, w.reshape(VOCAB * HID))
    logits_t = _tc_expand(ids, w[:, 0])
    return (logits_t.transpose(0, 2, 1),
            hidden_flat.reshape(BATCH, SEQ, HID))


# speedup vs baseline: 1.2182x; 1.2182x over previous
import functools

import jax
import jax.numpy as jnp
from jax import lax
from jax.experimental import pallas as pl
from jax.experimental.pallas import tpu as pltpu
from jax.experimental.pallas import tpu_sc as plsc

VOCAB = 64
HID = 8
BATCH = 32
SEQ = 8192
NTOK = BATCH * SEQ
NW = 32
TOK_PER_W = NTOK // NW
CHUNK = 2048
NCHUNK = TOK_PER_W // CHUNK
NGRP = CHUNK // 16


def _sc_body(ids_hbm, w_hbm, hidden_hbm,
             ids0, ids1, hid0, hid1, tmp_v, w_v,
             sem_i0, sem_i1, sem_o0, sem_o1):
    wid = lax.axis_index("s") * 2 + lax.axis_index("c")
    iota = lax.iota(jnp.int32, 16)
    lane_hi = jnp.right_shift(iota, 3)
    kpat = lax.bitwise_and(iota, 7)
    ids_b = (ids0, ids1)
    hid_b = (hid0, hid1)
    sem_i = (sem_i0, sem_i1)
    sem_o = (sem_o0, sem_o1)

    def tok0(c):
        return wid * TOK_PER_W + c * CHUNK

    pltpu.sync_copy(w_hbm, w_v)

    pltpu.async_copy(ids_hbm.at[pl.ds(tok0(0), CHUNK)], ids0, sem_i0)

    def do_chunk(c, b):
        nb = 1 - b
        pltpu.make_async_copy(
            ids_hbm.at[pl.ds(tok0(c), CHUNK)], ids_b[b], sem_i[b]).wait()

        @pl.when(c >= 2)
        def _():
            pltpu.make_async_copy(
                hid_b[b],
                hidden_hbm.at[pl.ds(tok0(c - 2) * HID, CHUNK * HID)],
                sem_o[b]).wait()

        @pl.when(c + 1 < NCHUNK)
        def _():
            pltpu.async_copy(ids_hbm.at[pl.ds(tok0(c + 1), CHUNK)],
                             ids_b[nb], sem_i[nb])

        def grp(g, carry):
            ids16 = ids_b[b][pl.ds(g * 16, 16)]
            tmp_v[...] = ids16 * HID
            for q in range(8):
                idv = plsc.load_gather(tmp_v, [lane_hi + 2 * q])
                hv = plsc.load_gather(w_v, [idv + kpat])
                hid_b[b][pl.ds(g * 128 + q * 16, 16)] = hv
            return carry

        lax.fori_loop(0, NGRP, grp, None)

        pltpu.async_copy(
            hid_b[b], hidden_hbm.at[pl.ds(tok0(c) * HID, CHUNK * HID)],
            sem_o[b])

    def loop_body(jj, carry):
        do_chunk(2 * jj, 0)
        do_chunk(2 * jj + 1, 1)
        return carry

    lax.fori_loop(0, NCHUNK // 2, loop_body, None)

    for c in (NCHUNK - 2, NCHUNK - 1):
        b = c % 2
        pltpu.make_async_copy(
            hid_b[b], hidden_hbm.at[pl.ds(tok0(c) * HID, CHUNK * HID)],
            sem_o[b]).wait()


@functools.partial(
    pl.kernel,
    out_type=[
        jax.ShapeDtypeStruct((NTOK * HID,), jnp.float32),
    ],
    mesh=plsc.VectorSubcoreMesh(core_axis_name="c", subcore_axis_name="s"),
    compiler_params=pltpu.CompilerParams(needs_layout_passes=False),
    scratch_types=[
        pltpu.VMEM((CHUNK,), jnp.int32),
        pltpu.VMEM((CHUNK,), jnp.int32),
        pltpu.VMEM((CHUNK * HID,), jnp.float32),
        pltpu.VMEM((CHUNK * HID,), jnp.float32),
        pltpu.VMEM((16,), jnp.int32),
        pltpu.VMEM((VOCAB * HID,), jnp.float32),
        pltpu.SemaphoreType.DMA,
        pltpu.SemaphoreType.DMA,
        pltpu.SemaphoreType.DMA,
        pltpu.SemaphoreType.DMA,
    ],
)
def _gather_sc(*refs):
    _sc_body(*refs)


TC_BS = 8192


def _tc_body(ids_ref, w0_ref, logits_ref):
    ids = ids_ref[...]
    w0 = w0_ref[...]
    iotav = lax.broadcasted_iota(jnp.int32, (VOCAB, TC_BS), 0)
    onehot = iotav == ids[None, :]
    h0 = jnp.where(onehot, w0[:, None], 0.0).sum(axis=0, keepdims=True)
    t = jnp.maximum(jnp.round(h0 * 10.0), 0.0)
    idx = t.astype(jnp.int32) % VOCAB
    vals = idx.astype(jnp.float32) / 10.0
    logits_ref[...] = jnp.where(iotav == idx, vals, 0.0)


_tc_expand = pl.pallas_call(
    _tc_body,
    grid=(BATCH,),
    in_specs=[
        pl.BlockSpec((TC_BS,), lambda b: (b,)),
        pl.BlockSpec((VOCAB,), lambda b: (0,)),
    ],
    out_specs=pl.BlockSpec((None, VOCAB, TC_BS), lambda b: (b, 0, 0)),
    out_shape=jax.ShapeDtypeStruct((BATCH, VOCAB, SEQ), jnp.float32),
    compiler_params=pltpu.CompilerParams(
        dimension_semantics=("parallel",)),
)


def kernel(input_ids, embedding_weight):
    ids = input_ids.astype(jnp.int32).reshape(NTOK)
    w = embedding_weight.astype(jnp.float32)
    (hidden_flat,) = _gather_sc(ids, w.reshape(VOCAB * HID))
    logits_t = _tc_expand(ids, w[:, 0])
    return (logits_t.transpose(0, 2, 1),
            hidden_flat.reshape(BATCH, SEQ, HID))


# speedup vs baseline: 1.2326x; 1.0118x over previous
import functools

import jax
import jax.numpy as jnp
from jax import lax
from jax.experimental import pallas as pl
from jax.experimental.pallas import tpu as pltpu
from jax.experimental.pallas import tpu_sc as plsc

VOCAB = 64
HID = 8
BATCH = 32
SEQ = 8192
NTOK = BATCH * SEQ
NW = 32
TOK_PER_W = NTOK // NW
CHUNK = 2048
NCHUNK = TOK_PER_W // CHUNK
NGRP = CHUNK // 16


def _sc_body(ids_hbm, w_hbm, hidden_hbm,
             ids0, ids1, hid0, hid1, tmp_v, w_v,
             sem_i0, sem_i1, sem_o0, sem_o1):
    wid = lax.axis_index("s") * 2 + lax.axis_index("c")
    iota = lax.iota(jnp.int32, 16)
    lane_hi = jnp.right_shift(iota, 3)
    kpat = lax.bitwise_and(iota, 7)
    ids_b = (ids0, ids1)
    hid_b = (hid0, hid1)
    sem_i = (sem_i0, sem_i1)
    sem_o = (sem_o0, sem_o1)

    def tok0(c):
        return wid * TOK_PER_W + c * CHUNK

    pltpu.sync_copy(w_hbm, w_v)

    pltpu.async_copy(ids_hbm.at[pl.ds(tok0(0), CHUNK)], ids0, sem_i0)

    def do_chunk(c, b):
        nb = 1 - b
        pltpu.make_async_copy(
            ids_hbm.at[pl.ds(tok0(c), CHUNK)], ids_b[b], sem_i[b]).wait()

        @pl.when(c >= 2)
        def _():
            pltpu.make_async_copy(
                hid_b[b],
                hidden_hbm.at[pl.ds(tok0(c - 2) * HID, CHUNK * HID)],
                sem_o[b]).wait()

        @pl.when(c + 1 < NCHUNK)
        def _():
            pltpu.async_copy(ids_hbm.at[pl.ds(tok0(c + 1), CHUNK)],
                             ids_b[nb], sem_i[nb])

        def grp(g, carry):
            ids16 = ids_b[b][pl.ds(g * 16, 16)]
            tmp_v[...] = ids16 * HID
            for q in range(8):
                idv = plsc.load_gather(tmp_v, [lane_hi + 2 * q])
                hv = plsc.load_gather(w_v, [idv + kpat])
                hid_b[b][pl.ds(g * 128 + q * 16, 16)] = hv
            return carry

        lax.fori_loop(0, NGRP, grp, None)

        pltpu.async_copy(
            hid_b[b], hidden_hbm.at[pl.ds(tok0(c) * HID, CHUNK * HID)],
            sem_o[b])

    def loop_body(jj, carry):
        do_chunk(2 * jj, 0)
        do_chunk(2 * jj + 1, 1)
        return carry

    lax.fori_loop(0, NCHUNK // 2, loop_body, None)

    for c in (NCHUNK - 2, NCHUNK - 1):
        b = c % 2
        pltpu.make_async_copy(
            hid_b[b], hidden_hbm.at[pl.ds(tok0(c) * HID, CHUNK * HID)],
            sem_o[b]).wait()


@functools.partial(
    pl.kernel,
    out_type=[
        jax.ShapeDtypeStruct((NTOK * HID,), jnp.float32),
    ],
    mesh=plsc.VectorSubcoreMesh(core_axis_name="c", subcore_axis_name="s"),
    compiler_params=pltpu.CompilerParams(needs_layout_passes=False),
    scratch_types=[
        pltpu.VMEM((CHUNK,), jnp.int32),
        pltpu.VMEM((CHUNK,), jnp.int32),
        pltpu.VMEM((CHUNK * HID,), jnp.float32),
        pltpu.VMEM((CHUNK * HID,), jnp.float32),
        pltpu.VMEM((16,), jnp.int32),
        pltpu.VMEM((VOCAB * HID,), jnp.float32),
        pltpu.SemaphoreType.DMA,
        pltpu.SemaphoreType.DMA,
        pltpu.SemaphoreType.DMA,
        pltpu.SemaphoreType.DMA,
    ],
)
def _gather_sc(*refs):
    _sc_body(*refs)


TC_BS = 8192


def _tc_body(ids_ref, w0_ref, logits_ref):
    ids = ids_ref[...]
    w0 = w0_ref[...]
    t = jnp.maximum(jnp.round(w0 * 10.0), 0.0)
    idx_tab = t.astype(jnp.int32) % VOCAB
    val_tab = idx_tab.astype(jnp.float32) / 10.0
    iotam = lax.broadcasted_iota(jnp.int32, (VOCAB, VOCAB), 0)
    m = jnp.where(iotam == idx_tab[None, :], val_tab[None, :], 0.0)
    iotav = lax.broadcasted_iota(jnp.int32, (VOCAB, TC_BS), 0)
    onehot = jnp.where(iotav == ids[None, :], 1.0, 0.0)
    logits_ref[...] = jax.lax.dot(m, onehot,
                                  preferred_element_type=jnp.float32)


_tc_expand = pl.pallas_call(
    _tc_body,
    grid=(BATCH,),
    in_specs=[
        pl.BlockSpec((TC_BS,), lambda b: (b,)),
        pl.BlockSpec((VOCAB,), lambda b: (0,)),
    ],
    out_specs=pl.BlockSpec((None, VOCAB, TC_BS), lambda b: (b, 0, 0)),
    out_shape=jax.ShapeDtypeStruct((BATCH, VOCAB, SEQ), jnp.float32),
    compiler_params=pltpu.CompilerParams(
        dimension_semantics=("parallel",)),
)


def kernel(input_ids, embedding_weight):
    ids = input_ids.astype(jnp.int32).reshape(NTOK)
    w = embedding_weight.astype(jnp.float32)
    (hidden_flat,) = _gather_sc(ids, w.reshape(VOCAB * HID))
    logits_t = _tc_expand(ids, w[:, 0])
    return (logits_t.transpose(0, 2, 1),
            hidden_flat.reshape(BATCH, SEQ, HID))
